# baseline (speedup 1.0000x reference)
import functools

import jax
import jax.numpy as jnp
from jax import lax
from jax.experimental import pallas as pl
from jax.experimental.pallas import tpu as pltpu
from jax.experimental.pallas import tpu_sc as plsc

N = 10000
D = 128
H = 128
OUT = 16
G = 64

NC = 2
NS = 16
NW = NC * NS
CHUNK = 128
CPW = 80
NBUF = 8
LA = 4
EPAD = NW * CPW * CHUNK
NPAD = 10240
RPT = NPAD // NS
RCH = RPT // CHUNK

_f32 = jnp.float32


def _sc_mesh():
  return plsc.VectorSubcoreMesh(
      core_axis_name="c", subcore_axis_name="s",
      num_cores=NC, num_subcores=NS)


HH = H // 2


@functools.cache
def _sc_agg_kernel():
  @functools.partial(
      pl.kernel,
      out_type=jax.ShapeDtypeStruct((NC * NPAD, H), _f32),
      mesh=_sc_mesh(),
      compiler_params=pltpu.CompilerParams(use_tc_tiling_on_sc=False),
      scratch_types=[
          pltpu.VMEM_SHARED((NPAD, HH), _f32),
          pltpu.VMEM((CPW, CHUNK), jnp.int32),
          pltpu.VMEM((CPW, CHUNK), jnp.int32),
          [pltpu.VMEM((CHUNK, HH), _f32) for _ in range(NBUF)],
          [pltpu.SemaphoreType.DMA for _ in range(NBUF)],
          [pltpu.SemaphoreType.DMA for _ in range(NBUF)],
      ],
  )
  def body(hlo_hbm, hhi_hbm, src_hbm, dst_hbm, z_hbm, out_hbm,
           acc, src_v, dst_v, rows, gsem, ssem):
    cid = lax.axis_index("c")
    sid = lax.axis_index("s")
    wid = sid * NC + cid

    pltpu.sync_copy(src_hbm.at[wid], src_v)
    pltpu.sync_copy(dst_hbm.at[wid], dst_v)

    for half, h_hbm in ((0, hlo_hbm), (1, hhi_hbm)):
      def fire_gather(c, b, h_hbm=h_hbm):
        pltpu.async_copy(h_hbm.at[src_v.at[c]], rows[b], gsem[b])

      def wait_gather(c, b, h_hbm=h_hbm):
        pltpu.make_async_copy(h_hbm.at[src_v.at[c]], rows[b], gsem[b]).wait()

      def fire_scatter(c, b):
        pltpu.async_copy(rows[b], acc.at[dst_v.at[c]], ssem[b], add=True)

      def wait_scatter(c, b):
        pltpu.make_async_copy(rows[b], acc.at[dst_v.at[c]], ssem[b]).wait()

      pltpu.sync_copy(z_hbm, rows[0])
      for j in range(RCH):
        pltpu.sync_copy(rows[0], acc.at[pl.ds(sid * RPT + j * CHUNK, CHUNK)])
      plsc.subcore_barrier()

      nsteady = (CPW - (NBUF - LA) - LA) // NBUF

      for c in range(LA):
        fire_gather(c, c % NBUF)
      for c in range(NBUF - LA):
        fire_gather(c + LA, (c + LA) % NBUF)
        wait_gather(c, c % NBUF)
        fire_scatter(c, c % NBUF)

      def superstep(s, carry):
        for j in range(NBUF):
          c = (NBUF - LA) + s * NBUF + j
          b = (NBUF - LA + j) % NBUF
          bl = j % NBUF
          wait_scatter(c + LA - NBUF, bl)
          fire_gather(c + LA, bl)
          wait_gather(c, b)
          fire_scatter(c, b)
        return carry

      lax.fori_loop(0, nsteady, superstep, 0, unroll=False)

      for c in range((NBUF - LA) + nsteady * NBUF, CPW - LA):
        wait_scatter(c + LA - NBUF, (c + LA) % NBUF)
        fire_gather(c + LA, (c + LA) % NBUF)
        wait_gather(c, c % NBUF)
        fire_scatter(c, c % NBUF)
      for c in range(CPW - LA, CPW):
        wait_scatter(c + LA - NBUF, (c + LA) % NBUF)
        wait_gather(c, c % NBUF)
        fire_scatter(c, c % NBUF)
      for c in range(CPW - NBUF + LA, CPW):
        wait_scatter(c, c % NBUF)

      plsc.subcore_barrier()
      for j in range(RCH):
        r0 = sid * RPT + j * CHUNK
        pltpu.sync_copy(acc.at[pl.ds(r0, CHUNK)], rows[0])
        pltpu.sync_copy(
            rows[0],
            out_hbm.at[pl.ds(cid * NPAD + r0, CHUNK),
                       pl.ds(half * HH, HH)])
      if half == 0:
        plsc.subcore_barrier()

  return body


def _sc_agg(h_lo, h_hi, src_p, dst_p, z64):
  return _sc_agg_kernel()(h_lo, h_hi, src_p, dst_p, z64)


_DS = 4
DW = 16


@functools.cache
def _sc_deg_kernel():
  @functools.partial(
      pl.kernel,
      out_type=jax.ShapeDtypeStruct((NC * NPAD, H), _f32),
      mesh=_sc_mesh(),
      compiler_params=pltpu.CompilerParams(use_tc_tiling_on_sc=False),
      scratch_types=[
          pltpu.VMEM_SHARED((NPAD, DW), _f32),
          pltpu.VMEM((CPW, CHUNK), jnp.int32),
          pltpu.VMEM((CHUNK, DW), _f32),
          [pltpu.SemaphoreType.DMA for _ in range(_DS)],
      ],
  )
  def body(dst_hbm, z_hbm, one_hbm, out_hbm, acc, dst_v, rows_v, ssem):
    cid = lax.axis_index("c")
    sid = lax.axis_index("s")
    wid = sid * NC + cid
    pltpu.sync_copy(z_hbm, rows_v)
    for j in range(RCH):
      pltpu.sync_copy(rows_v, acc.at[pl.ds(sid * RPT + j * CHUNK, CHUNK)])
    pltpu.sync_copy(one_hbm, rows_v)
    pltpu.sync_copy(dst_hbm.at[wid], dst_v)
    plsc.subcore_barrier()

    def fire(c, b):
      pltpu.async_copy(rows_v, acc.at[dst_v.at[c]], ssem[b], add=True)

    def drain(c, b):
      pltpu.make_async_copy(rows_v, acc.at[dst_v.at[c]], ssem[b]).wait()

    for c in range(_DS):
      fire(c, c % _DS)

    def superstep(s, carry):
      for j in range(_DS):
        c = _DS + s * _DS + j
        drain(c - _DS, j)
        fire(c, j)
      return carry

    lax.fori_loop(0, (CPW - _DS) // _DS, superstep, 0, unroll=False)
    for c in range(CPW - _DS, CPW):
      drain(c, c % _DS)

    plsc.subcore_barrier()
    for j in range(RCH):
      r0 = sid * RPT + j * CHUNK
      pltpu.sync_copy(acc.at[pl.ds(r0, CHUNK)], rows_v)
      pltpu.sync_copy(rows_v, out_hbm.at[pl.ds(cid * NPAD + r0, CHUNK),
                                         pl.ds(0, DW)])

  return body


def _sc_deg(dst_p, z16, one16):
  return _sc_deg_kernel()(dst_p, z16, one16)


_BR = 2000


def _mlp_body(x_ref, w1_ref, b1_ref, w2_ref, b2_ref, olo_ref, ohi_ref):
  x = x_ref[...]
  t = jnp.maximum(
      jnp.dot(x, w1_ref[...], preferred_element_type=_f32) + b1_ref[...], 0.0)
  o = jnp.dot(t, w2_ref[...], preferred_element_type=_f32) + b2_ref[...]
  olo_ref[...] = o[:, :HH]
  ohi_ref[...] = o[:, HH:]


def _mlp(nodes, w1, b1, w2, b2):
  grid = N // _BR
  return pl.pallas_call(
      _mlp_body,
      grid=(grid,),
      in_specs=[
          pl.BlockSpec((_BR, D), lambda i: (i, 0)),
          pl.BlockSpec((D, H), lambda i: (0, 0)),
          pl.BlockSpec((1, H), lambda i: (0, 0)),
          pl.BlockSpec((H, H), lambda i: (0, 0)),
          pl.BlockSpec((1, H), lambda i: (0, 0)),
      ],
      out_specs=[pl.BlockSpec((_BR, HH), lambda i: (i, 0)),
                 pl.BlockSpec((_BR, HH), lambda i: (i, 0))],
      out_shape=[jax.ShapeDtypeStruct((N, HH), _f32),
                 jax.ShapeDtypeStruct((N, HH), _f32)],
  )(nodes, w1, b1, w2, b2)


def _layer_b(hlo_ref, hhi_ref, aa_ref, ab_ref,
             da_ref, db_ref, ws_ref, wn_ref, b_ref, g_ref, be_ref):
  deg = jnp.maximum(da_ref[...] + db_ref[...], 1.0)
  h = jnp.concatenate([hlo_ref[...], hhi_ref[...]], axis=1)
  agg = (aa_ref[...] + ab_ref[...]) / deg
  r = (jnp.dot(h, ws_ref[...], preferred_element_type=_f32)
       + jnp.dot(agg, wn_ref[...], preferred_element_type=_f32)
       + b_ref[...])
  r = jnp.maximum(r, 0.0)
  mu = jnp.mean(r, axis=-1, keepdims=True)
  var = jnp.mean((r - mu) ** 2, axis=-1, keepdims=True)
  return (r - mu) * lax.rsqrt(var + 1e-5) * g_ref[...] + be_ref[...]


def _layer_body(hlo_ref, hhi_ref, aa_ref, ab_ref,
                da_ref, db_ref, ws_ref, wn_ref, b_ref, g_ref, be_ref,
                olo_ref, ohi_ref):
  o = _layer_b(hlo_ref, hhi_ref, aa_ref, ab_ref,
               da_ref, db_ref, ws_ref, wn_ref, b_ref, g_ref, be_ref)
  olo_ref[...] = o[:, :HH]
  ohi_ref[...] = o[:, HH:]


_LAYER_IN_SPECS = [
    pl.BlockSpec((_BR, HH), lambda i: (i, 0)),
    pl.BlockSpec((_BR, HH), lambda i: (i, 0)),
    pl.BlockSpec((_BR, H), lambda i: (i, 0)),
    pl.BlockSpec((_BR, H), lambda i: (i, 0)),
    pl.BlockSpec((_BR, 1), lambda i: (i, 0)),
    pl.BlockSpec((_BR, 1), lambda i: (i, 0)),
    pl.BlockSpec((H, H), lambda i: (0, 0)),
    pl.BlockSpec((H, H), lambda i: (0, 0)),
    pl.BlockSpec((1, H), lambda i: (0, 0)),
    pl.BlockSpec((1, H), lambda i: (0, 0)),
    pl.BlockSpec((1, H), lambda i: (0, 0)),
]


def _layer(hlo, hhi, aa, ab, deg_a, deg_b, ws, wn, b, g, be):
  grid = N // _BR
  return pl.pallas_call(
      _layer_body,
      grid=(grid,),
      in_specs=_LAYER_IN_SPECS,
      out_specs=[pl.BlockSpec((_BR, HH), lambda i: (i, 0)),
                 pl.BlockSpec((_BR, HH), lambda i: (i, 0))],
      out_shape=[jax.ShapeDtypeStruct((N, HH), _f32),
                 jax.ShapeDtypeStruct((N, HH), _f32)],
  )(hlo, hhi, aa, ab, deg_a, deg_b, ws, wn, b, g, be)


def _layer_pool_body(hlo_ref, hhi_ref, aa_ref, ab_ref,
                     da_ref, db_ref, ws_ref, wn_ref, b_ref, g_ref, be_ref,
                     gid_ref, wo_ref, bo_ref, o_ref, acc_ref, cnt_ref):
  i = pl.program_id(0)

  @pl.when(i == 0)
  def _():
    acc_ref[...] = jnp.zeros_like(acc_ref)
    cnt_ref[...] = jnp.zeros_like(cnt_ref)

  o = _layer_b(hlo_ref, hhi_ref, aa_ref, ab_ref,
               da_ref, db_ref, ws_ref, wn_ref, b_ref, g_ref, be_ref)
  onehot = (gid_ref[...] ==
            lax.broadcasted_iota(jnp.int32, (1, G), 1)).astype(_f32)
  acc_ref[...] += lax.dot_general(onehot, o, (((0,), (0,)), ((), ())),
                                  preferred_element_type=_f32)
  cnt_ref[...] += lax.dot_general(onehot, jnp.ones((_BR, 1), _f32),
                                  (((0,), (0,)), ((), ())),
                                  preferred_element_type=_f32)

  @pl.when(i == (N // _BR) - 1)
  def _():
    pooled = acc_ref[...] / jnp.maximum(cnt_ref[...], 1.0)
    o_ref[...] = jnp.dot(pooled, wo_ref[...],
                         preferred_element_type=_f32) + bo_ref[...]


def _layer_pool(hlo, hhi, aa, ab, deg_a, deg_b, ws, wn, b, g, be,
                gid2d, wo, bo):
  grid = N // _BR
  return pl.pallas_call(
      _layer_pool_body,
      grid=(grid,),
      in_specs=_LAYER_IN_SPECS + [
          pl.BlockSpec((_BR, 1), lambda i: (i, 0)),
          pl.BlockSpec((H, OUT), lambda i: (0, 0)),
          pl.BlockSpec((1, OUT), lambda i: (0, 0)),
      ],
      out_specs=pl.BlockSpec((G, OUT), lambda i: (0, 0)),
      out_shape=jax.ShapeDtypeStruct((G, OUT), _f32),
      scratch_shapes=[
          pltpu.VMEM((G, H), _f32),
          pltpu.VMEM((G, 1), _f32),
      ],
  )(hlo, hhi, aa, ab, deg_a, deg_b, ws, wn, b, g, be,
    gid2d, wo, bo)


def kernel(nodes, edge_index, graph_ids,
           W_in1, b_in1, W_in2, b_in2,
           W_self_0, W_neigh_0, bias_0, ln_g_0, ln_b_0,
           W_self_1, W_neigh_1, bias_1, ln_g_1, ln_b_1,
           W_self_2, W_neigh_2, bias_2, ln_g_2, ln_b_2,
           W_out, b_out):
  E = edge_index.shape[1]
  src = edge_index[0]
  dst = edge_index[1]
  pad_n = EPAD - E
  pad_i = jnp.arange(pad_n, dtype=jnp.int32)
  src_p = jnp.concatenate([src, (pad_i * 97) % N]).reshape(NW, CPW, CHUNK)
  dst_p = jnp.concatenate(
      [dst, N + (pad_i % (NPAD - N))]).reshape(NW, CPW, CHUNK)

  z16 = jnp.zeros((CHUNK, DW), _f32)
  one16 = jnp.ones((CHUNK, DW), _f32)
  degp = _sc_deg(dst_p, z16, one16)
  z64 = jnp.zeros((CHUNK, HH), _f32)
  deg_a = lax.slice(degp, (0, 0), (N, 1))
  deg_b = lax.slice(degp, (NPAD, 0), (NPAD + N, 1))

  h_lo, h_hi = _mlp(nodes, W_in1, b_in1.reshape(1, H),
                    W_in2, b_in2.reshape(1, H))

  layer_params = [
      (W_self_0, W_neigh_0, bias_0, ln_g_0, ln_b_0),
      (W_self_1, W_neigh_1, bias_1, ln_g_1, ln_b_1),
      (W_self_2, W_neigh_2, bias_2, ln_g_2, ln_b_2),
  ]
  for li, (ws, wn, b, g, be) in enumerate(layer_params):
    aggp = _sc_agg(h_lo, h_hi, src_p, dst_p, z64)
    agg_a = lax.slice(aggp, (0, 0), (N, H))
    agg_b = lax.slice(aggp, (NPAD, 0), (NPAD + N, H))
    args = (h_lo, h_hi, agg_a, agg_b, deg_a, deg_b,
            ws, wn, b.reshape(1, H), g.reshape(1, H), be.reshape(1, H))
    if li < 2:
      h_lo, h_hi = _layer(*args)
    else:
      return _layer_pool(*args, graph_ids.reshape(N, 1),
                         W_out, b_out.reshape(1, OUT))


# speedup vs baseline: 1.0359x; 1.0359x over previous
import functools

import jax
import jax.numpy as jnp
from jax import lax
from jax.experimental import pallas as pl
from jax.experimental.pallas import tpu as pltpu
from jax.experimental.pallas import tpu_sc as plsc

N = 10000
D = 128
H = 128
OUT = 16
G = 64

NC = 2
NS = 16
NW = NC * NS
CHUNK = 128
CPW = 80
NBUF = 6
LA = 3
EPAD = NW * CPW * CHUNK
NPAD = 10240
RPT = NPAD // NS
RCH = RPT // CHUNK

_f32 = jnp.float32


def _sc_mesh():
  return plsc.VectorSubcoreMesh(
      core_axis_name="c", subcore_axis_name="s",
      num_cores=NC, num_subcores=NS)


HH = H // 2


@functools.cache
def _sc_agg_kernel():
  @functools.partial(
      pl.kernel,
      out_type=jax.ShapeDtypeStruct((NC * NPAD, H), _f32),
      mesh=_sc_mesh(),
      compiler_params=pltpu.CompilerParams(use_tc_tiling_on_sc=False),
      scratch_types=[
          pltpu.VMEM_SHARED((NPAD, HH), _f32),
          pltpu.VMEM((CPW, CHUNK), jnp.int32),
          pltpu.VMEM((CPW, CHUNK), jnp.int32),
          [pltpu.VMEM((CHUNK, HH), _f32) for _ in range(NBUF)],
          [pltpu.SemaphoreType.DMA for _ in range(NBUF)],
          [pltpu.SemaphoreType.DMA for _ in range(NBUF)],
      ],
  )
  def body(hlo_hbm, hhi_hbm, src_hbm, dst_hbm, z_hbm, out_hbm,
           acc, src_v, dst_v, rows, gsem, ssem):
    cid = lax.axis_index("c")
    sid = lax.axis_index("s")
    wid = sid * NC + cid

    pltpu.sync_copy(src_hbm.at[wid], src_v)
    pltpu.sync_copy(dst_hbm.at[wid], dst_v)

    for half, h_hbm in ((0, hlo_hbm), (1, hhi_hbm)):
      def fire_gather(c, b, h_hbm=h_hbm):
        pltpu.async_copy(h_hbm.at[src_v.at[c]], rows[b], gsem[b])

      def wait_gather(c, b, h_hbm=h_hbm):
        pltpu.make_async_copy(h_hbm.at[src_v.at[c]], rows[b], gsem[b]).wait()

      def fire_scatter(c, b):
        pltpu.async_copy(rows[b], acc.at[dst_v.at[c]], ssem[b], add=True)

      def wait_scatter(c, b):
        pltpu.make_async_copy(rows[b], acc.at[dst_v.at[c]], ssem[b]).wait()

      pltpu.sync_copy(z_hbm, rows[0])
      for j in range(RCH):
        pltpu.sync_copy(rows[0], acc.at[pl.ds(sid * RPT + j * CHUNK, CHUNK)])
      plsc.subcore_barrier()

      nsteady = (CPW - (NBUF - LA) - LA) // NBUF

      for c in range(LA):
        fire_gather(c, c % NBUF)
      for c in range(NBUF - LA):
        fire_gather(c + LA, (c + LA) % NBUF)
        wait_gather(c, c % NBUF)
        fire_scatter(c, c % NBUF)

      def superstep(s, carry):
        for j in range(NBUF):
          c = (NBUF - LA) + s * NBUF + j
          b = (NBUF - LA + j) % NBUF
          bl = j % NBUF
          wait_scatter(c + LA - NBUF, bl)
          fire_gather(c + LA, bl)
          wait_gather(c, b)
          fire_scatter(c, b)
        return carry

      lax.fori_loop(0, nsteady, superstep, 0, unroll=False)

      for c in range((NBUF - LA) + nsteady * NBUF, CPW - LA):
        wait_scatter(c + LA - NBUF, (c + LA) % NBUF)
        fire_gather(c + LA, (c + LA) % NBUF)
        wait_gather(c, c % NBUF)
        fire_scatter(c, c % NBUF)
      for c in range(CPW - LA, CPW):
        wait_scatter(c + LA - NBUF, (c + LA) % NBUF)
        wait_gather(c, c % NBUF)
        fire_scatter(c, c % NBUF)
      for c in range(CPW - NBUF + LA, CPW):
        wait_scatter(c, c % NBUF)

      plsc.subcore_barrier()
      for j in range(RCH):
        r0 = sid * RPT + j * CHUNK
        pltpu.sync_copy(acc.at[pl.ds(r0, CHUNK)], rows[0])
        pltpu.sync_copy(
            rows[0],
            out_hbm.at[pl.ds(cid * NPAD + r0, CHUNK),
                       pl.ds(half * HH, HH)])
      if half == 0:
        plsc.subcore_barrier()

  return body


def _sc_agg(h_lo, h_hi, src_p, dst_p, z64):
  return _sc_agg_kernel()(h_lo, h_hi, src_p, dst_p, z64)


_DS = 4
DW = 16


@functools.cache
def _sc_deg_kernel():
  @functools.partial(
      pl.kernel,
      out_type=jax.ShapeDtypeStruct((NC * NPAD, H), _f32),
      mesh=_sc_mesh(),
      compiler_params=pltpu.CompilerParams(use_tc_tiling_on_sc=False),
      scratch_types=[
          pltpu.VMEM_SHARED((NPAD, DW), _f32),
          pltpu.VMEM((CPW, CHUNK), jnp.int32),
          pltpu.VMEM((CHUNK, DW), _f32),
          [pltpu.SemaphoreType.DMA for _ in range(_DS)],
      ],
  )
  def body(dst_hbm, z_hbm, one_hbm, out_hbm, acc, dst_v, rows_v, ssem):
    cid = lax.axis_index("c")
    sid = lax.axis_index("s")
    wid = sid * NC + cid
    pltpu.sync_copy(z_hbm, rows_v)
    for j in range(RCH):
      pltpu.sync_copy(rows_v, acc.at[pl.ds(sid * RPT + j * CHUNK, CHUNK)])
    pltpu.sync_copy(one_hbm, rows_v)
    pltpu.sync_copy(dst_hbm.at[wid], dst_v)
    plsc.subcore_barrier()

    def fire(c, b):
      pltpu.async_copy(rows_v, acc.at[dst_v.at[c]], ssem[b], add=True)

    def drain(c, b):
      pltpu.make_async_copy(rows_v, acc.at[dst_v.at[c]], ssem[b]).wait()

    for c in range(_DS):
      fire(c, c % _DS)

    def superstep(s, carry):
      for j in range(_DS):
        c = _DS + s * _DS + j
        drain(c - _DS, j)
        fire(c, j)
      return carry

    lax.fori_loop(0, (CPW - _DS) // _DS, superstep, 0, unroll=False)
    for c in range(CPW - _DS, CPW):
      drain(c, c % _DS)

    plsc.subcore_barrier()
    for j in range(RCH):
      r0 = sid * RPT + j * CHUNK
      pltpu.sync_copy(acc.at[pl.ds(r0, CHUNK)], rows_v)
      pltpu.sync_copy(rows_v, out_hbm.at[pl.ds(cid * NPAD + r0, CHUNK),
                                         pl.ds(0, DW)])

  return body


def _sc_deg(dst_p, z16, one16):
  return _sc_deg_kernel()(dst_p, z16, one16)


_BR = 2000


def _mlp_body(x_ref, w1_ref, b1_ref, w2_ref, b2_ref, olo_ref, ohi_ref):
  x = x_ref[...]
  t = jnp.maximum(
      jnp.dot(x, w1_ref[...], preferred_element_type=_f32) + b1_ref[...], 0.0)
  o = jnp.dot(t, w2_ref[...], preferred_element_type=_f32) + b2_ref[...]
  olo_ref[...] = o[:, :HH]
  ohi_ref[...] = o[:, HH:]


def _mlp(nodes, w1, b1, w2, b2):
  grid = N // _BR
  return pl.pallas_call(
      _mlp_body,
      grid=(grid,),
      in_specs=[
          pl.BlockSpec((_BR, D), lambda i: (i, 0)),
          pl.BlockSpec((D, H), lambda i: (0, 0)),
          pl.BlockSpec((1, H), lambda i: (0, 0)),
          pl.BlockSpec((H, H), lambda i: (0, 0)),
          pl.BlockSpec((1, H), lambda i: (0, 0)),
      ],
      out_specs=[pl.BlockSpec((_BR, HH), lambda i: (i, 0)),
                 pl.BlockSpec((_BR, HH), lambda i: (i, 0))],
      out_shape=[jax.ShapeDtypeStruct((N, HH), _f32),
                 jax.ShapeDtypeStruct((N, HH), _f32)],
  )(nodes, w1, b1, w2, b2)


def _layer_b(hlo_ref, hhi_ref, aa_ref, ab_ref,
             da_ref, db_ref, ws_ref, wn_ref, b_ref, g_ref, be_ref):
  deg = jnp.maximum(da_ref[...] + db_ref[...], 1.0)
  h = jnp.concatenate([hlo_ref[...], hhi_ref[...]], axis=1)
  agg = (aa_ref[...] + ab_ref[...]) / deg
  r = (jnp.dot(h, ws_ref[...], preferred_element_type=_f32)
       + jnp.dot(agg, wn_ref[...], preferred_element_type=_f32)
       + b_ref[...])
  r = jnp.maximum(r, 0.0)
  mu = jnp.mean(r, axis=-1, keepdims=True)
  var = jnp.mean((r - mu) ** 2, axis=-1, keepdims=True)
  return (r - mu) * lax.rsqrt(var + 1e-5) * g_ref[...] + be_ref[...]


def _layer_body(hlo_ref, hhi_ref, aa_ref, ab_ref,
                da_ref, db_ref, ws_ref, wn_ref, b_ref, g_ref, be_ref,
                olo_ref, ohi_ref):
  o = _layer_b(hlo_ref, hhi_ref, aa_ref, ab_ref,
               da_ref, db_ref, ws_ref, wn_ref, b_ref, g_ref, be_ref)
  olo_ref[...] = o[:, :HH]
  ohi_ref[...] = o[:, HH:]


_LAYER_IN_SPECS = [
    pl.BlockSpec((_BR, HH), lambda i: (i, 0)),
    pl.BlockSpec((_BR, HH), lambda i: (i, 0)),
    pl.BlockSpec((_BR, H), lambda i: (i, 0)),
    pl.BlockSpec((_BR, H), lambda i: (i, 0)),
    pl.BlockSpec((_BR, 1), lambda i: (i, 0)),
    pl.BlockSpec((_BR, 1), lambda i: (i, 0)),
    pl.BlockSpec((H, H), lambda i: (0, 0)),
    pl.BlockSpec((H, H), lambda i: (0, 0)),
    pl.BlockSpec((1, H), lambda i: (0, 0)),
    pl.BlockSpec((1, H), lambda i: (0, 0)),
    pl.BlockSpec((1, H), lambda i: (0, 0)),
]


def _layer(hlo, hhi, aa, ab, deg_a, deg_b, ws, wn, b, g, be):
  grid = N // _BR
  return pl.pallas_call(
      _layer_body,
      grid=(grid,),
      in_specs=_LAYER_IN_SPECS,
      out_specs=[pl.BlockSpec((_BR, HH), lambda i: (i, 0)),
                 pl.BlockSpec((_BR, HH), lambda i: (i, 0))],
      out_shape=[jax.ShapeDtypeStruct((N, HH), _f32),
                 jax.ShapeDtypeStruct((N, HH), _f32)],
  )(hlo, hhi, aa, ab, deg_a, deg_b, ws, wn, b, g, be)


def _layer_pool_body(hlo_ref, hhi_ref, aa_ref, ab_ref,
                     da_ref, db_ref, ws_ref, wn_ref, b_ref, g_ref, be_ref,
                     gid_ref, wo_ref, bo_ref, o_ref, acc_ref, cnt_ref):
  i = pl.program_id(0)

  @pl.when(i == 0)
  def _():
    acc_ref[...] = jnp.zeros_like(acc_ref)
    cnt_ref[...] = jnp.zeros_like(cnt_ref)

  o = _layer_b(hlo_ref, hhi_ref, aa_ref, ab_ref,
               da_ref, db_ref, ws_ref, wn_ref, b_ref, g_ref, be_ref)
  onehot = (gid_ref[...] ==
            lax.broadcasted_iota(jnp.int32, (1, G), 1)).astype(_f32)
  acc_ref[...] += lax.dot_general(onehot, o, (((0,), (0,)), ((), ())),
                                  preferred_element_type=_f32)
  cnt_ref[...] += lax.dot_general(onehot, jnp.ones((_BR, 1), _f32),
                                  (((0,), (0,)), ((), ())),
                                  preferred_element_type=_f32)

  @pl.when(i == (N // _BR) - 1)
  def _():
    pooled = acc_ref[...] / jnp.maximum(cnt_ref[...], 1.0)
    o_ref[...] = jnp.dot(pooled, wo_ref[...],
                         preferred_element_type=_f32) + bo_ref[...]


def _layer_pool(hlo, hhi, aa, ab, deg_a, deg_b, ws, wn, b, g, be,
                gid2d, wo, bo):
  grid = N // _BR
  return pl.pallas_call(
      _layer_pool_body,
      grid=(grid,),
      in_specs=_LAYER_IN_SPECS + [
          pl.BlockSpec((_BR, 1), lambda i: (i, 0)),
          pl.BlockSpec((H, OUT), lambda i: (0, 0)),
          pl.BlockSpec((1, OUT), lambda i: (0, 0)),
      ],
      out_specs=pl.BlockSpec((G, OUT), lambda i: (0, 0)),
      out_shape=jax.ShapeDtypeStruct((G, OUT), _f32),
      scratch_shapes=[
          pltpu.VMEM((G, H), _f32),
          pltpu.VMEM((G, 1), _f32),
      ],
  )(hlo, hhi, aa, ab, deg_a, deg_b, ws, wn, b, g, be,
    gid2d, wo, bo)


def kernel(nodes, edge_index, graph_ids,
           W_in1, b_in1, W_in2, b_in2,
           W_self_0, W_neigh_0, bias_0, ln_g_0, ln_b_0,
           W_self_1, W_neigh_1, bias_1, ln_g_1, ln_b_1,
           W_self_2, W_neigh_2, bias_2, ln_g_2, ln_b_2,
           W_out, b_out):
  E = edge_index.shape[1]
  src = edge_index[0]
  dst = edge_index[1]
  pad_n = EPAD - E
  pad_i = jnp.arange(pad_n, dtype=jnp.int32)
  src_p = jnp.concatenate([src, (pad_i * 97) % N]).reshape(NW, CPW, CHUNK)
  dst_p = jnp.concatenate(
      [dst, N + (pad_i % (NPAD - N))]).reshape(NW, CPW, CHUNK)

  z16 = jnp.zeros((CHUNK, DW), _f32)
  one16 = jnp.ones((CHUNK, DW), _f32)
  degp = _sc_deg(dst_p, z16, one16)
  z64 = jnp.zeros((CHUNK, HH), _f32)
  deg_a = lax.slice(degp, (0, 0), (N, 1))
  deg_b = lax.slice(degp, (NPAD, 0), (NPAD + N, 1))

  h_lo, h_hi = _mlp(nodes, W_in1, b_in1.reshape(1, H),
                    W_in2, b_in2.reshape(1, H))

  layer_params = [
      (W_self_0, W_neigh_0, bias_0, ln_g_0, ln_b_0),
      (W_self_1, W_neigh_1, bias_1, ln_g_1, ln_b_1),
      (W_self_2, W_neigh_2, bias_2, ln_g_2, ln_b_2),
  ]
  for li, (ws, wn, b, g, be) in enumerate(layer_params):
    aggp = _sc_agg(h_lo, h_hi, src_p, dst_p, z64)
    agg_a = lax.slice(aggp, (0, 0), (N, H))
    agg_b = lax.slice(aggp, (NPAD, 0), (NPAD + N, H))
    args = (h_lo, h_hi, agg_a, agg_b, deg_a, deg_b,
            ws, wn, b.reshape(1, H), g.reshape(1, H), be.reshape(1, H))
    if li < 2:
      h_lo, h_hi = _layer(*args)
    else:
      return _layer_pool(*args, graph_ids.reshape(N, 1),
                         W_out, b_out.reshape(1, OUT))
